# baseline (speedup 1.0000x reference)
import functools

import jax
import jax.numpy as jnp
from jax.experimental import pallas as pl
from jax.experimental.pallas import tpu as pltpu

_A = 64
_T = 128
_EQU = 3
_INV = 16
_H = 64
_NL = 2
_NN = _A * _T
_P = _A * _A
_LOG_SQRT_2PI = 0.9189385332046727


def _hsilu(t):
    return t * jnp.tanh(t) + t


def _egnn_block_kernel(x0_ref, hin_ref, eps_ref, sisj_ref, smask_ref,
                       embwt_ref, embb_ref,
                       a1t_ref, b1t_ref, c1_ref, w2t_ref, be2_ref,
                       wc1t_ref, bc1_ref, wc2_ref, wn1at_ref, wn1bt_ref,
                       bn1_ref, wn2t_ref, bn2_ref, logstd_ref,
                       act_ref, lp_ref):
    f32 = jnp.float32
    bf16 = jnp.bfloat16
    x = x0_ref[0]
    h = jnp.dot(hin_ref[...], embwt_ref[...],
                preferred_element_type=f32) + embb_ref[...]
    sisj = sisj_ref[...]
    smask = smask_ref[...]

    for l in range(_NL):
        cd = x[:, :, None] - x[:, None, :]
        radial = jnp.sum(cd * cd, axis=0)
        norm = jnp.sqrt(radial) + 1e-8
        cdn = cd / norm

        hA = jnp.dot(h, a1t_ref[l], preferred_element_type=f32) + c1_ref[l, 1:]
        hB = jnp.dot(h, b1t_ref[l], preferred_element_type=f32)
        hab = jnp.concatenate([hA, hB], axis=0).astype(bf16)
        pre = (jnp.dot(sisj, hab, preferred_element_type=f32)
                   .reshape(_A, _A, _H)
               + radial[:, :, None] * c1_ref[l, 0])
        m1 = _hsilu(pre).reshape(_P, _H).astype(bf16)
        m2 = _hsilu(jnp.dot(m1, w2t_ref[l],
                            preferred_element_type=f32) + be2_ref[l])
        m2 = m2.astype(bf16)
        cm = _hsilu(jnp.dot(m2, wc1t_ref[l],
                            preferred_element_type=f32) + bc1_ref[l])
        csc = jnp.tanh(jnp.dot(cm.astype(bf16), wc2_ref[l],
                               preferred_element_type=f32)
                       .reshape(_A, _A))
        s = jnp.sum(cdn * csc, axis=-1)
        x = x + s * (1.0 / 63.0)

        am = jnp.dot(smask, m2, preferred_element_type=f32)
        o = _hsilu(jnp.dot(h, wn1at_ref[l], preferred_element_type=f32)
                   + jnp.dot(am, wn1bt_ref[l], preferred_element_type=f32)
                   + bn1_ref[l])
        h = h + jnp.dot(o, wn2t_ref[l], preferred_element_type=f32) + bn2_ref[l]

    e = eps_ref[0]
    logstd = logstd_ref[...]
    act_ref[0] = x + jnp.exp(logstd) * e
    lp_ref[0] = -0.5 * (e * e) - logstd - _LOG_SQRT_2PI


@jax.jit
def _run(x0, hin, eps, consts):
    def blk(shape):
        nd = len(shape)
        return pl.BlockSpec(shape, lambda i, _n=nd: (0,) * _n)

    in_specs = [
        pl.BlockSpec((1, _EQU, _A), lambda i: (i, 0, 0)),
        pl.BlockSpec((_A, _INV), lambda i: (i, 0)),
        pl.BlockSpec((1, _EQU, _A), lambda i: (i, 0, 0)),
    ] + [blk(w.shape) for w in consts]
    out_specs = [pl.BlockSpec((1, _EQU, _A), lambda i: (i, 0, 0))] * 2
    out_shape = [jax.ShapeDtypeStruct((_T, _EQU, _A), jnp.float32)] * 2

    act, lp = pl.pallas_call(
        _egnn_block_kernel,
        grid=(_T,),
        in_specs=in_specs,
        out_specs=out_specs,
        out_shape=out_shape,
        compiler_params=pltpu.CompilerParams(
            dimension_semantics=("arbitrary",)),
    )(x0, hin, eps, *consts)
    return act, lp


def kernel(obs, rnn_states, masks, params, row, col, eps):
    L = params['layers']
    H = _H

    def stk(f):
        return jnp.stack([f(p) for p in L])

    pr = jnp.arange(_P, dtype=jnp.int32)
    ia = jnp.arange(_A, dtype=jnp.int32)
    si = (pr[:, None] // _A == ia[None, :]).astype(jnp.float32)
    sj = (pr[:, None] % _A == ia[None, :]).astype(jnp.float32)
    sisj = jnp.concatenate([si, sj], axis=1)
    smask = (si * (1.0 - sj)).T

    bf16 = jnp.bfloat16
    consts = [
        sisj.astype(bf16), smask.astype(bf16),
        params['emb_W'].T,
        params['emb_b'].reshape(1, H),
        stk(lambda p: 0.5 * p['We1'][:, :H].T),
        stk(lambda p: 0.5 * p['We1'][:, H:2 * H].T),
        stk(lambda p: jnp.stack([0.5 * p['We1'][:, 2 * H],
                                 0.5 * p['be1']])),
        stk(lambda p: (0.5 * p['We2'].T).astype(bf16)),
        stk(lambda p: 0.5 * p['be2'].reshape(1, H)),
        stk(lambda p: (0.5 * p['Wc1'].T).astype(bf16)),
        stk(lambda p: 0.5 * p['bc1'].reshape(1, H)),
        stk(lambda p: p['Wc2'].reshape(H, 1).astype(bf16)),
        stk(lambda p: 0.5 * p['Wn1'][:, :H].T),
        stk(lambda p: 0.5 * p['Wn1'][:, H:].T),
        stk(lambda p: 0.5 * p['bn1'].reshape(1, H)),
        stk(lambda p: p['Wn2'].T),
        stk(lambda p: p['bn2'].reshape(1, H)),
        params['log_std'].reshape(_EQU, 1),
    ]
    x0 = obs[:, :_EQU].reshape(_T, _A, _EQU).transpose(0, 2, 1)
    epsT = eps.reshape(_T, _A, _EQU).transpose(0, 2, 1)
    hin = obs[:, _EQU:]
    act, lp = _run(x0, hin, epsT, consts)
    return (act.transpose(0, 2, 1),
            lp.transpose(0, 2, 1),
            rnn_states)


# speedup vs baseline: 1.1693x; 1.1693x over previous
import functools

import jax
import jax.numpy as jnp
from jax.experimental import pallas as pl
from jax.experimental.pallas import tpu as pltpu

_A = 64
_T = 128
_EQU = 3
_INV = 16
_H = 64
_NL = 2
_NN = _A * _T
_P = _A * _A
_LOG_SQRT_2PI = 0.9189385332046727


def _hsilu(t):
    return t * jnp.tanh(t) + t


def _egnn_block_kernel(x0_ref, hin_ref, eps_ref, sisj_ref, smask_ref,
                       embwt_ref, embb_ref,
                       a1t_ref, b1t_ref, c1_ref, w2t_ref, be2_ref,
                       wc1t_ref, bc1_ref, wc2_ref, wn1at_ref, wn1bt_ref,
                       bn1_ref, wn2t_ref, bn2_ref, logstd_ref,
                       act_ref, lp_ref):
    f32 = jnp.float32
    bf16 = jnp.bfloat16
    x = x0_ref[0]
    h = jnp.dot(hin_ref[...], embwt_ref[...],
                preferred_element_type=f32) + embb_ref[...]
    sisj = sisj_ref[...]
    smask = smask_ref[...]

    for l in range(_NL):
        cd = x[:, :, None] - x[:, None, :]
        radial = jnp.sum(cd * cd, axis=0)
        norm = jnp.sqrt(radial) + 1e-8
        cdn = cd / norm

        hA = jnp.dot(h, a1t_ref[l], preferred_element_type=f32) + c1_ref[l, 1:]
        hB = jnp.dot(h, b1t_ref[l], preferred_element_type=f32)
        hab = jnp.concatenate([hA, hB], axis=0).astype(bf16)
        pre = (jnp.dot(sisj, hab, preferred_element_type=f32)
                   .reshape(_A, _A, _H)
               + radial[:, :, None] * c1_ref[l, 0])
        m1 = _hsilu(pre).reshape(_P, _H).astype(bf16)
        m2 = _hsilu(jnp.dot(m1, w2t_ref[l],
                            preferred_element_type=f32) + be2_ref[l])
        m2 = m2.astype(bf16)
        cm = _hsilu(jnp.dot(m2, wc1t_ref[l],
                            preferred_element_type=f32) + bc1_ref[l])
        csc = jnp.tanh(jnp.sum(cm.reshape(_A, _A, _H) * wc2_ref[l],
                               axis=-1))
        s = jnp.sum(cdn * csc, axis=1)
        x = x - s * (1.0 / 63.0)

        am = jnp.dot(smask, m2, preferred_element_type=f32)
        o = _hsilu(jnp.dot(h, wn1at_ref[l], preferred_element_type=f32)
                   + jnp.dot(am, wn1bt_ref[l], preferred_element_type=f32)
                   + bn1_ref[l])
        h = h + jnp.dot(o, wn2t_ref[l], preferred_element_type=f32) + bn2_ref[l]

    e = eps_ref[0]
    logstd = logstd_ref[...]
    act_ref[0] = x + jnp.exp(logstd) * e
    lp_ref[0] = -0.5 * (e * e) - logstd - _LOG_SQRT_2PI


@jax.jit
def _run(x0, hin, eps, consts):
    def blk(shape):
        nd = len(shape)
        return pl.BlockSpec(shape, lambda i, _n=nd: (0,) * _n)

    in_specs = [
        pl.BlockSpec((1, _EQU, _A), lambda i: (i, 0, 0)),
        pl.BlockSpec((_A, _INV), lambda i: (i, 0)),
        pl.BlockSpec((1, _EQU, _A), lambda i: (i, 0, 0)),
    ] + [blk(w.shape) for w in consts]
    out_specs = [pl.BlockSpec((1, _EQU, _A), lambda i: (i, 0, 0))] * 2
    out_shape = [jax.ShapeDtypeStruct((_T, _EQU, _A), jnp.float32)] * 2

    act, lp = pl.pallas_call(
        _egnn_block_kernel,
        grid=(_T,),
        in_specs=in_specs,
        out_specs=out_specs,
        out_shape=out_shape,
        compiler_params=pltpu.CompilerParams(
            dimension_semantics=("arbitrary",)),
    )(x0, hin, eps, *consts)
    return act, lp


def kernel(obs, rnn_states, masks, params, row, col, eps):
    L = params['layers']
    H = _H

    def stk(f):
        return jnp.stack([f(p) for p in L])

    pr = jnp.arange(_P, dtype=jnp.int32)
    ia = jnp.arange(_A, dtype=jnp.int32)
    si = (pr[:, None] % _A == ia[None, :]).astype(jnp.float32)
    sj = (pr[:, None] // _A == ia[None, :]).astype(jnp.float32)
    sisj = jnp.concatenate([si, sj], axis=1)
    smask = (si * (1.0 - sj)).T

    bf16 = jnp.bfloat16
    consts = [
        sisj.astype(bf16), smask.astype(bf16),
        params['emb_W'].T,
        params['emb_b'].reshape(1, H),
        stk(lambda p: 0.5 * p['We1'][:, :H].T),
        stk(lambda p: 0.5 * p['We1'][:, H:2 * H].T),
        stk(lambda p: jnp.stack([0.5 * p['We1'][:, 2 * H],
                                 0.5 * p['be1']])),
        stk(lambda p: (0.5 * p['We2'].T).astype(bf16)),
        stk(lambda p: 0.5 * p['be2'].reshape(1, H)),
        stk(lambda p: (0.5 * p['Wc1'].T).astype(bf16)),
        stk(lambda p: 0.5 * p['bc1'].reshape(1, H)),
        stk(lambda p: p['Wc2'].reshape(1, H)),
        stk(lambda p: 0.5 * p['Wn1'][:, :H].T),
        stk(lambda p: 0.5 * p['Wn1'][:, H:].T),
        stk(lambda p: 0.5 * p['bn1'].reshape(1, H)),
        stk(lambda p: p['Wn2'].T),
        stk(lambda p: p['bn2'].reshape(1, H)),
        params['log_std'].reshape(_EQU, 1),
    ]
    x0 = obs[:, :_EQU].reshape(_T, _A, _EQU).transpose(0, 2, 1)
    epsT = eps.reshape(_T, _A, _EQU).transpose(0, 2, 1)
    hin = obs[:, _EQU:]
    act, lp = _run(x0, hin, epsT, consts)
    return (act.transpose(0, 2, 1),
            lp.transpose(0, 2, 1),
            rnn_states)


# speedup vs baseline: 1.2135x; 1.0379x over previous
import functools

import jax
import jax.numpy as jnp
from jax.experimental import pallas as pl
from jax.experimental.pallas import tpu as pltpu

_A = 64
_T = 128
_EQU = 3
_INV = 16
_H = 64
_NL = 2
_NN = _A * _T
_P = _A * _A
_LOG_SQRT_2PI = 0.9189385332046727


def _hsilu(t):
    return t * jnp.tanh(t) + t


def _egnn_block_kernel(x0_ref, hin_ref, eps_ref, sisj_ref, smask_ref,
                       embwt_ref, embb_ref,
                       a1t_ref, b1t_ref, c1_ref, w2t_ref, be2_ref,
                       wc1t_ref, bc1_ref, wc2_ref, wn1at_ref, wn1bt_ref,
                       bn1_ref, wn2t_ref, bn2_ref, logstd_ref,
                       act_ref, lp_ref):
    f32 = jnp.float32
    bf16 = jnp.bfloat16
    x = x0_ref[0]
    h = jnp.dot(hin_ref[...], embwt_ref[...],
                preferred_element_type=f32) + embb_ref[...]
    sisj = sisj_ref[...]
    smask = smask_ref[...]

    for l in range(_NL):
        cd = x[:, :, None] - x[:, None, :]
        radial = jnp.sum(cd * cd, axis=0)
        norm = jnp.sqrt(radial) + 1e-8
        cdn = cd / norm

        hA = jnp.dot(h, a1t_ref[l], preferred_element_type=f32) + c1_ref[l, 1:]
        hB = jnp.dot(h, b1t_ref[l], preferred_element_type=f32)
        hab = jnp.concatenate([hA, hB], axis=0).astype(bf16)
        pre = (jnp.dot(sisj, hab, preferred_element_type=f32)
                   .reshape(_A, _A, _H)
               + radial[:, :, None] * c1_ref[l, 0]).astype(bf16)
        m1 = _hsilu(pre).reshape(_P, _H)
        m2 = _hsilu((jnp.dot(m1, w2t_ref[l], preferred_element_type=f32)
                     + be2_ref[l]).astype(bf16))
        cm = _hsilu(jnp.dot(m2, wc1t_ref[l], preferred_element_type=f32)
                    + bc1_ref[l])
        csc = jnp.tanh(jnp.sum(cm.reshape(_A, _A, _H) * wc2_ref[l],
                               axis=-1))
        s = jnp.sum(cdn * csc, axis=1)
        x = x - s * (1.0 / 63.0)

        am = jnp.dot(smask, m2, preferred_element_type=f32)
        o = _hsilu(jnp.dot(h, wn1at_ref[l], preferred_element_type=f32)
                   + jnp.dot(am, wn1bt_ref[l], preferred_element_type=f32)
                   + bn1_ref[l])
        h = h + jnp.dot(o, wn2t_ref[l], preferred_element_type=f32) + bn2_ref[l]

    e = eps_ref[0]
    logstd = logstd_ref[...]
    act_ref[0] = x + jnp.exp(logstd) * e
    lp_ref[0] = -0.5 * (e * e) - logstd - _LOG_SQRT_2PI


@jax.jit
def _run(x0, hin, eps, consts):
    def blk(shape):
        nd = len(shape)
        return pl.BlockSpec(shape, lambda i, _n=nd: (0,) * _n)

    in_specs = [
        pl.BlockSpec((1, _EQU, _A), lambda i: (i, 0, 0)),
        pl.BlockSpec((_A, _INV), lambda i: (i, 0)),
        pl.BlockSpec((1, _EQU, _A), lambda i: (i, 0, 0)),
    ] + [blk(w.shape) for w in consts]
    out_specs = [pl.BlockSpec((1, _EQU, _A), lambda i: (i, 0, 0))] * 2
    out_shape = [jax.ShapeDtypeStruct((_T, _EQU, _A), jnp.float32)] * 2

    act, lp = pl.pallas_call(
        _egnn_block_kernel,
        grid=(_T,),
        in_specs=in_specs,
        out_specs=out_specs,
        out_shape=out_shape,
        compiler_params=pltpu.CompilerParams(
            dimension_semantics=("arbitrary",)),
    )(x0, hin, eps, *consts)
    return act, lp


def kernel(obs, rnn_states, masks, params, row, col, eps):
    L = params['layers']
    H = _H

    def stk(f):
        return jnp.stack([f(p) for p in L])

    pr = jnp.arange(_P, dtype=jnp.int32)
    ia = jnp.arange(_A, dtype=jnp.int32)
    si = (pr[:, None] % _A == ia[None, :]).astype(jnp.float32)
    sj = (pr[:, None] // _A == ia[None, :]).astype(jnp.float32)
    sisj = jnp.concatenate([si, sj], axis=1)
    smask = (si * (1.0 - sj)).T

    bf16 = jnp.bfloat16
    consts = [
        sisj.astype(bf16), smask.astype(bf16),
        params['emb_W'].T,
        params['emb_b'].reshape(1, H),
        stk(lambda p: 0.5 * p['We1'][:, :H].T),
        stk(lambda p: 0.5 * p['We1'][:, H:2 * H].T),
        stk(lambda p: jnp.stack([0.5 * p['We1'][:, 2 * H],
                                 0.5 * p['be1']])),
        stk(lambda p: (0.5 * p['We2'].T).astype(bf16)),
        stk(lambda p: 0.5 * p['be2'].reshape(1, H)),
        stk(lambda p: (0.5 * p['Wc1'].T).astype(bf16)),
        stk(lambda p: 0.5 * p['bc1'].reshape(1, H)),
        stk(lambda p: p['Wc2'].reshape(1, H)),
        stk(lambda p: 0.5 * p['Wn1'][:, :H].T),
        stk(lambda p: 0.5 * p['Wn1'][:, H:].T),
        stk(lambda p: 0.5 * p['bn1'].reshape(1, H)),
        stk(lambda p: p['Wn2'].T),
        stk(lambda p: p['bn2'].reshape(1, H)),
        params['log_std'].reshape(_EQU, 1),
    ]
    x0 = obs[:, :_EQU].reshape(_T, _A, _EQU).transpose(0, 2, 1)
    epsT = eps.reshape(_T, _A, _EQU).transpose(0, 2, 1)
    hin = obs[:, _EQU:]
    act, lp = _run(x0, hin, epsT, consts)
    return (act.transpose(0, 2, 1),
            lp.transpose(0, 2, 1),
            rnn_states)


# speedup vs baseline: 1.7207x; 1.4179x over previous
import functools

import jax
import jax.numpy as jnp
from jax.experimental import pallas as pl
from jax.experimental.pallas import tpu as pltpu

_A = 64
_T = 128
_EQU = 3
_INV = 16
_H = 64
_NL = 2
_NN = _A * _T
_P = _A * _A
_B = 2
_LOG_SQRT_2PI = 0.9189385332046727


def _hsilu(t):
    return t * jnp.tanh(t) + t


def _egnn_block_kernel(x0_ref, hin_ref, eps_ref, sisj_ref, smask_ref,
                       sdif_ref, embw_ref, embb_ref,
                       a1_ref, b1_ref, c1_ref, w2_ref, be2_ref,
                       wc1_ref, bc1_ref, wc2_ref, wn1a_ref, wn1b_ref,
                       bn1_ref, wn2_ref, bn2_ref, logstd_ref,
                       act_ref, lp_ref):
    f32 = jnp.float32
    bf16 = jnp.bfloat16
    sisj = sisj_ref[...]
    smask = smask_ref[...]
    sdif = sdif_ref[...]

    xs = [x0_ref[t] for t in range(_B)]
    hs = [jnp.dot(embw_ref[...], hin_ref[t],
                  preferred_element_type=f32) + embb_ref[...]
          for t in range(_B)]

    for t in range(_B):
      x = xs[t]
      h = hs[t]
      for l in range(_NL):
        cdf = jnp.dot(x.astype(bf16), sdif,
                      preferred_element_type=f32)
        radial = jnp.sum(cdf * cdf, axis=0, keepdims=True)
        inorm = 1.0 / (jnp.sqrt(radial) + 1e-8)
        cdn = cdf * inorm

        hA = jnp.dot(a1_ref[l], h, preferred_element_type=f32) + c1_ref[l, :, 1:2]
        hB = jnp.dot(b1_ref[l], h, preferred_element_type=f32)
        hab = jnp.concatenate([hA, hB], axis=1).astype(bf16)
        pre = (jnp.dot(hab, sisj, preferred_element_type=f32).astype(bf16)
               + radial.astype(bf16) * c1_ref[l, :, 0:1].astype(bf16))
        m1 = _hsilu(pre)
        m2 = _hsilu(jnp.dot(w2_ref[l], m1, preferred_element_type=f32)
                    .astype(bf16) + be2_ref[l])
        cm = _hsilu(jnp.dot(wc1_ref[l], m2, preferred_element_type=f32)
                    .astype(bf16) + bc1_ref[l])
        csc = jnp.tanh(jnp.sum(cm * wc2_ref[l], axis=0,
                               keepdims=True).astype(f32))
        trans = (cdn * csc).reshape(_EQU, _A, _A)
        s = jnp.sum(trans, axis=1)
        x = x - s * (1.0 / 63.0)

        am = jnp.dot(m2, smask, preferred_element_type=f32)
        o = _hsilu(jnp.dot(wn1a_ref[l], h, preferred_element_type=f32)
                   + jnp.dot(wn1b_ref[l], am, preferred_element_type=f32)
                   + bn1_ref[l])
        h = h + jnp.dot(wn2_ref[l], o, preferred_element_type=f32) + bn2_ref[l]

      e = eps_ref[t]
      logstd = logstd_ref[...]
      act_ref[t] = x + jnp.exp(logstd) * e
      lp_ref[t] = -0.5 * (e * e) - logstd - _LOG_SQRT_2PI


@jax.jit
def _run(x0, hin, eps, consts):
    def blk(shape):
        nd = len(shape)
        return pl.BlockSpec(shape, lambda i, _n=nd: (0,) * _n)

    in_specs = [
        pl.BlockSpec((_B, _EQU, _A), lambda i: (i, 0, 0)),
        pl.BlockSpec((_B, _INV, _A), lambda i: (i, 0, 0)),
        pl.BlockSpec((_B, _EQU, _A), lambda i: (i, 0, 0)),
    ] + [blk(w.shape) for w in consts]
    out_specs = [pl.BlockSpec((_B, _EQU, _A), lambda i: (i, 0, 0))] * 2
    out_shape = [jax.ShapeDtypeStruct((_T, _EQU, _A), jnp.float32)] * 2

    act, lp = pl.pallas_call(
        _egnn_block_kernel,
        grid=(_T // _B,),
        in_specs=in_specs,
        out_specs=out_specs,
        out_shape=out_shape,
        compiler_params=pltpu.CompilerParams(
            dimension_semantics=("arbitrary",)),
    )(x0, hin, eps, *consts)
    return act, lp


def kernel(obs, rnn_states, masks, params, row, col, eps):
    L = params['layers']
    H = _H
    bf16 = jnp.bfloat16

    def stk(f):
        return jnp.stack([f(p) for p in L])

    pr = jnp.arange(_P, dtype=jnp.int32)
    ia = jnp.arange(_A, dtype=jnp.int32)
    si = (pr[:, None] % _A == ia[None, :]).astype(jnp.float32)
    sj = (pr[:, None] // _A == ia[None, :]).astype(jnp.float32)
    sisj = jnp.concatenate([si, sj], axis=1)
    smask = si * (1.0 - sj)

    consts = [
        sisj.T.astype(bf16),
        smask.astype(bf16),
        (sj - si).T.astype(bf16),
        params['emb_W'],
        params['emb_b'].reshape(H, 1),
        stk(lambda p: 0.5 * p['We1'][:, :H]),
        stk(lambda p: 0.5 * p['We1'][:, H:2 * H]),
        stk(lambda p: jnp.stack([0.5 * p['We1'][:, 2 * H],
                                 0.5 * p['be1']], axis=1)),
        stk(lambda p: (0.5 * p['We2']).astype(bf16)),
        stk(lambda p: (0.5 * p['be2'].reshape(H, 1)).astype(bf16)),
        stk(lambda p: (0.5 * p['Wc1']).astype(bf16)),
        stk(lambda p: (0.5 * p['bc1'].reshape(H, 1)).astype(bf16)),
        stk(lambda p: p['Wc2'].reshape(H, 1).astype(bf16)),
        stk(lambda p: 0.5 * p['Wn1'][:, :H]),
        stk(lambda p: 0.5 * p['Wn1'][:, H:]),
        stk(lambda p: 0.5 * p['bn1'].reshape(H, 1)),
        stk(lambda p: p['Wn2']),
        stk(lambda p: p['bn2'].reshape(H, 1)),
        params['log_std'].reshape(_EQU, 1),
    ]
    x0 = obs[:, :_EQU].reshape(_T, _A, _EQU).transpose(0, 2, 1)
    hinT = obs[:, _EQU:].reshape(_T, _A, _INV).transpose(0, 2, 1)
    epsT = eps.reshape(_T, _A, _EQU).transpose(0, 2, 1)
    act, lp = _run(x0, hinT, epsT, consts)
    return (act.transpose(0, 2, 1),
            lp.transpose(0, 2, 1),
            rnn_states)
